# baseline (speedup 1.0000x reference)
import jax
import jax.numpy as jnp
from jax.experimental import pallas as pl
from jax.experimental.pallas import tpu as pltpu

_TILE = 256


def _round_up(n, m):
    return ((n + m - 1) // m) * m


def _ncf_body(idx_ref,
              g0_ref, g1_ref,
              m0_ref, m1_ref,
              w1a_ref, w1b_ref,
              b1_ref, w2_ref, b2_ref,
              wg_ref, wm_ref,
              bfc_ref,
              out_ref,
              ag0, ag1, am0, am1):
    for m in range(_TILE):
        i0 = idx_ref[m, 0]
        i1 = idx_ref[m, 1]
        ag0[m] = g0_ref[i0, 0]
        ag1[m] = g1_ref[i1, 0]
        am0[m] = m0_ref[i0, 0]
        am1[m] = m1_ref[i1, 0]

    A0 = am0[...]
    A1 = am1[...]
    gmf = ag0[...] * ag1[...]

    h = (jnp.dot(A0, w1a_ref[...], preferred_element_type=jnp.float32)
         + jnp.dot(A1, w1b_ref[...], preferred_element_type=jnp.float32)
         + b1_ref[...])
    h = jnp.maximum(h, 0.0)
    h = jnp.dot(h, w2_ref[...], preferred_element_type=jnp.float32) + b2_ref[...]
    h = jnp.maximum(h, 0.0)

    logit = (jnp.sum(gmf * wg_ref[...], axis=-1, keepdims=True)
             + jnp.sum(h * wm_ref[...], axis=-1, keepdims=True)
             + bfc_ref[0, 0])
    out_ref[...] = jax.nn.sigmoid(logit)


def kernel(x, gmf_t0, gmf_t1, mlp_t0, mlp_t1, w1, b1, w2, b2, wfc, bfc):
    B = x.shape[0]
    E = gmf_t0.shape[1]

    b_pad = _round_up(max(B, 1), _TILE)
    num_tiles = b_pad // _TILE

    idx = x.astype(jnp.int32)
    if b_pad != B:
        idx = jnp.pad(idx, ((0, b_pad - B), (0, 0)))

    g0 = gmf_t0.reshape(-1, 1, E)
    g1 = gmf_t1.reshape(-1, 1, E)
    m0 = mlp_t0.reshape(-1, 1, E)
    m1 = mlp_t1.reshape(-1, 1, E)

    w1a, w1b = w1[:E, :], w1[E:, :]
    wg = wfc[:E, :].T
    wm = wfc[E:, :].T

    def resident(a):
        return pl.BlockSpec(a.shape, lambda g: (0,) * a.ndim)

    flops = 2 * b_pad * (E * 128 * 2 + 128 * 64) + b_pad * (4 * E + 4 * 64)
    bytes_accessed = 4 * gmf_t0.size * 4 + b_pad * (2 * 4 + 4 * E * 4 + 4)
    out = pl.pallas_call(
        _ncf_body,
        out_shape=jax.ShapeDtypeStruct((b_pad, 1), jnp.float32),
        grid=(num_tiles,),
        in_specs=[
            pl.BlockSpec((_TILE, 2), lambda g: (g, 0),
                         memory_space=pltpu.MemorySpace.SMEM),
            resident(g0), resident(g1), resident(m0), resident(m1),
            resident(w1a), resident(w1b), resident(b1),
            resident(w2), resident(b2),
            resident(wg), resident(wm),
            pl.BlockSpec(memory_space=pltpu.MemorySpace.SMEM),
        ],
        out_specs=pl.BlockSpec((_TILE, 1), lambda g: (g, 0)),
        scratch_shapes=[
            pltpu.VMEM((_TILE, E), jnp.float32),
            pltpu.VMEM((_TILE, E), jnp.float32),
            pltpu.VMEM((_TILE, E), jnp.float32),
            pltpu.VMEM((_TILE, E), jnp.float32),
        ],
        compiler_params=pltpu.CompilerParams(
            dimension_semantics=("parallel",)),
        cost_estimate=pl.CostEstimate(flops=flops, transcendentals=b_pad,
                                      bytes_accessed=bytes_accessed),
    )(idx, g0, g1, m0, m1, w1a, w1b, b1, w2, b2, wg, wm, bfc)
    return out[:B]
